# baseline (speedup 1.0000x reference)
import jax
import jax.numpy as jnp
from jax import lax
from jax.experimental import pallas as pl
from jax.experimental.pallas import tpu as pltpu
from jax.experimental.pallas import tpu_sc as plsc

N = 10000
E = 640000
FEAT = 1582
H = 64
R = 12
NR = N * R

NC, NS, L = 2, 16, 16
NW = NC * NS
CH = 128
SUB = 2
SED = SUB * CH
SUPW = 80
EPAD = NW * SUPW * SED
PAD = EPAD - E
NCHUNK = EPAD // CH
NRT = 120320
NRT_W = NRT // NW
NACC = N + 16
ACC_T = NACC // NS
CNT_T = NRT // NS

_MESH = plsc.VectorSubcoreMesh(
    core_axis_name="c", subcore_axis_name="s", num_cores=NC, num_subcores=NS)
_SC_PARAMS = pltpu.CompilerParams(use_tc_tiling_on_sc=False)


def _leaky(x):
    return jnp.where(x > 0, x, 0.01 * x)


def _mm(a, b):
    return lax.dot_general(a, b, (((1,), (0,)), ((), ())),
                           precision=lax.Precision.HIGHEST,
                           preferred_element_type=jnp.float32)


def _mmT(at, b):
    return lax.dot_general(at, b, (((0,), (0,)), ((), ())),
                           precision=lax.Precision.HIGHEST,
                           preferred_element_type=jnp.float32)


_NB = 10
_BN = N // _NB


def _enc_body(ft_ref, wenc_ref, benc_ref, wi_ref, bi_ref, wrelf_ref,
              wroot_ref, br_ref, y_ref, root_ref):
    a = _leaky(_mmT(ft_ref[...], wenc_ref[...]) + benc_ref[...])
    x = _leaky(_mm(a, wi_ref[...]) + bi_ref[...])
    y_ref[...] = _mm(x, wrelf_ref[...])
    root_ref[...] = _mm(x, wroot_ref[...]) + br_ref[...]


def _mid_body(p0_ref, p1_ref, root_ref, wrelf_ref, wroot_ref, br_ref,
              y_ref, root2_ref):
    x = p0_ref[...] + p1_ref[...] + root_ref[...]
    y_ref[...] = _mm(x, wrelf_ref[...])
    root2_ref[...] = _mm(x, wroot_ref[...]) + br_ref[...]


def _out_body(p0_ref, p1_ref, root_ref, wo1_ref, bo1_ref, wo2_ref, bo2_ref,
              o_ref):
    x = p0_ref[...] + p1_ref[...] + root_ref[...]
    h = _leaky(_mm(x, wo1_ref[...]) + bo1_ref[...])
    o_ref[...] = _mm(h, wo2_ref[...]) + bo2_ref[...]


def _full(shape):
    return pl.BlockSpec(shape, lambda i: tuple(0 for _ in shape))


def _rows(cols):
    return pl.BlockSpec((_BN, cols), lambda i: (i, 0))


_BNE = 1024


def _tc_encoder(ft, wenc, benc, wi, bi, wrelf, wroot, br):
    return pl.pallas_call(
        _enc_body,
        grid=(pl.cdiv(N, _BNE),),
        in_specs=[pl.BlockSpec((FEAT, _BNE), lambda i: (0, i)),
                  _full((FEAT, H)), _full((1, H)),
                  _full((H, H)), _full((1, H)), _full((H, R * H)),
                  _full((H, H)), _full((1, H))],
        out_specs=[pl.BlockSpec((_BNE, R * H), lambda i: (i, 0)),
                   pl.BlockSpec((_BNE, H), lambda i: (i, 0))],
        out_shape=[jax.ShapeDtypeStruct((N, R * H), jnp.float32),
                   jax.ShapeDtypeStruct((N, H), jnp.float32)],
    )(ft, wenc, benc, wi, bi, wrelf, wroot, br)


def _tc_mid(p0, p1, root, wrelf, wroot, br):
    return pl.pallas_call(
        _mid_body,
        grid=(_NB,),
        in_specs=[_rows(H), _rows(H), _rows(H), _full((H, R * H)),
                  _full((H, H)), _full((1, H))],
        out_specs=[_rows(R * H), _rows(H)],
        out_shape=[jax.ShapeDtypeStruct((N, R * H), jnp.float32),
                   jax.ShapeDtypeStruct((N, H), jnp.float32)],
    )(p0, p1, root, wrelf, wroot, br)


def _tc_out(p0, p1, root, wo1, bo1, wo2p, bo2p):
    return pl.pallas_call(
        _out_body,
        grid=(_NB,),
        in_specs=[_rows(H), _rows(H), _rows(H), _full((H, H)),
                  _full((1, H)), _full((H, 128)), _full((1, 128))],
        out_specs=[_rows(128)],
        out_shape=[jax.ShapeDtypeStruct((N, 128), jnp.float32)],
    )(p0, p1, root, wo1, bo1, wo2p, bo2p)[0]


def _wid():
    return lax.axis_index("s") * NC + lax.axis_index("c")


def _idx_from_packed(pk, g2, gd2, d2):
    for j in range(SUB):
        for k in range(CH // L):
            f = pl.ds(j * CH + k * L, L)
            s = pl.ds(k * L, L)
            sv = pk[0, f]
            dv = pk[1, f]
            ev = pk[2, f]
            g2[j, s] = sv * R + ev
            gd2[j, s] = dv * R + ev
            d2[j, s] = dv


def _sc_counts_body(src_hbm, dst_hbm, et_hbm, zeros1_hbm, c0_hbm, c1_hbm, *sc):
    pk = sc[0:2]
    ci = sc[2:4]
    gdum = sc[4:6]
    ddum = sc[6:8]
    ones_v = sc[8]
    acc = sc[9]
    psem = sc[10:12]
    cid = lax.axis_index("c")
    sid = lax.axis_index("s")
    wid = _wid()
    base = wid * SUPW

    pltpu.sync_copy(zeros1_hbm.at[pl.ds(sid * CNT_T, CNT_T)],
                    acc.at[pl.ds(sid * CNT_T, CNT_T)])
    for k in range(CH // L):
        ones_v[pl.ds(k * L, L)] = jnp.full((L,), 1.0, jnp.float32)
    plsc.subcore_barrier()

    def fire_packed(s_, p):
        pltpu.async_copy(src_hbm.at[pl.ds(s_ * SED, SED)], pk[p].at[0],
                         psem[p])
        pltpu.async_copy(dst_hbm.at[pl.ds(s_ * SED, SED)], pk[p].at[1],
                         psem[p])
        pltpu.async_copy(et_hbm.at[pl.ds(s_ * SED, SED)], pk[p].at[2],
                         psem[p])

    def wait_packed(s_, p):
        pltpu.make_async_copy(src_hbm.at[pl.ds(s_ * SED, SED)], pk[p].at[0],
                              psem[p]).wait()
        pltpu.make_async_copy(dst_hbm.at[pl.ds(s_ * SED, SED)], pk[p].at[1],
                              psem[p]).wait()
        pltpu.make_async_copy(et_hbm.at[pl.ds(s_ * SED, SED)], pk[p].at[2],
                              psem[p]).wait()

    for p in range(2):
        fire_packed(base + p, p)

    def body(t, carry):
        for u in range(2):
            s = base + 2 * t + u
            wait_packed(s, u)
            _idx_from_packed(pk[u], gdum[u], ci[u], ddum[u])

            @pl.when(s + 2 < base + SUPW)
            def _():
                fire_packed(s + 2, u)
            for j in range(SUB):
                pltpu.sync_copy(ones_v, acc.at[ci[u].at[j]], add=True)
        return carry

    lax.fori_loop(0, SUPW // 2, body, 0)
    plsc.subcore_barrier()

    @pl.when(cid == 0)
    def _():
        pltpu.sync_copy(acc.at[pl.ds(sid * CNT_T, CNT_T)],
                        c0_hbm.at[pl.ds(sid * CNT_T, CNT_T)])

    @pl.when(cid == 1)
    def _():
        pltpu.sync_copy(acc.at[pl.ds(sid * CNT_T, CNT_T)],
                        c1_hbm.at[pl.ds(sid * CNT_T, CNT_T)])


def _sc_counts(srcp, dstp, etp, zeros1):
    return pl.kernel(
        _sc_counts_body,
        out_type=[jax.ShapeDtypeStruct((NRT,), jnp.float32),
                  jax.ShapeDtypeStruct((NRT,), jnp.float32)],
        mesh=_MESH,
        compiler_params=_SC_PARAMS,
        scratch_types=(
            [pltpu.VMEM((3, SED), jnp.int32)] * 2
            + [pltpu.VMEM((SUB, CH), jnp.int32)] * 6
            + [pltpu.VMEM((CH,), jnp.float32),
               pltpu.VMEM_SHARED((NRT,), jnp.float32)]
            + [pltpu.SemaphoreType.DMA] * 2
        ),
    )(srcp, dstp, etp, zeros1)


def _sc_inv_body(c0_hbm, c1_hbm, invc_hbm, c0_v, c1_v, iv_v, sem):
    wid = _wid()
    off = wid * NRT_W
    pltpu.sync_copy(c0_hbm.at[pl.ds(off, NRT_W)], c0_v)
    pltpu.sync_copy(c1_hbm.at[pl.ds(off, NRT_W)], c1_v)

    def body(k, carry):
        s = pl.ds(k * L, L)
        iv_v[s] = 1.0 / (c0_v[s] + c1_v[s])
        return carry

    lax.fori_loop(0, NRT_W // L, body, 0)
    pltpu.sync_copy(iv_v, invc_hbm.at[pl.ds(off, NRT_W)])


def _sc_inv(c0, c1):
    return pl.kernel(
        _sc_inv_body,
        out_type=[jax.ShapeDtypeStruct((NRT,), jnp.float32)],
        mesh=_MESH,
        compiler_params=_SC_PARAMS,
        scratch_types=[
            pltpu.VMEM((NRT_W,), jnp.float32),
            pltpu.VMEM((NRT_W,), jnp.float32),
            pltpu.VMEM((NRT_W,), jnp.float32),
            pltpu.SemaphoreType.DMA,
        ],
    )(c0, c1)[0]


def _sc_edge_body(src_hbm, dst_hbm, et_hbm, invc_hbm, y_hbm, zeros2_hbm,
                  p0_hbm, p1_hbm, *sc):
    pk = sc[0:4]
    g2 = sc[4:8]
    gd2 = sc[8:12]
    d2 = sc[12:16]
    w2 = sc[16:20]
    rows = sc[20:24]
    acc = sc[24]
    psem = sc[25:29]
    gsem = sc[29:33]
    cid = lax.axis_index("c")
    sid = lax.axis_index("s")
    wid = _wid()
    base = wid * SUPW
    last = base + SUPW

    pltpu.sync_copy(zeros2_hbm.at[pl.ds(sid * ACC_T, ACC_T)],
                    acc.at[pl.ds(sid * ACC_T, ACC_T)])
    plsc.subcore_barrier()

    def fire_packed(s, p):
        pltpu.async_copy(src_hbm.at[pl.ds(s * SED, SED)], pk[p].at[0],
                         psem[p])
        pltpu.async_copy(dst_hbm.at[pl.ds(s * SED, SED)], pk[p].at[1],
                         psem[p])
        pltpu.async_copy(et_hbm.at[pl.ds(s * SED, SED)], pk[p].at[2],
                         psem[p])

    def wait_packed(s, p):
        pltpu.make_async_copy(src_hbm.at[pl.ds(s * SED, SED)], pk[p].at[0],
                              psem[p]).wait()
        pltpu.make_async_copy(dst_hbm.at[pl.ds(s * SED, SED)], pk[p].at[1],
                              psem[p]).wait()
        pltpu.make_async_copy(et_hbm.at[pl.ds(s * SED, SED)], pk[p].at[2],
                              psem[p]).wait()

    def fire_gathers(p):
        for j in range(SUB):
            pltpu.async_copy(y_hbm.at[g2[p].at[j]],
                             rows[p].at[pl.ds(j * CH, CH)], gsem[p])
            pltpu.async_copy(invc_hbm.at[gd2[p].at[j]], w2[p].at[j], gsem[p])

    def drain_gathers(p):
        for j in range(SUB):
            pltpu.make_async_copy(y_hbm.at[g2[p].at[j]],
                                  rows[p].at[pl.ds(j * CH, CH)],
                                  gsem[p]).wait()
            pltpu.make_async_copy(invc_hbm.at[gd2[p].at[j]], w2[p].at[j],
                                  gsem[p]).wait()

    def scale(p):
        rp, wp = rows[p], w2[p]
        for j in range(SUB):
            def kb(k, carry, _j=j):
                w16 = wp[_j, pl.ds(k * L, L)]
                for l in range(L):
                    ws = w16[l]
                    ri = _j * CH + k * L + l
                    for cc in range(H // L):
                        s2 = pl.ds(cc * L, L)
                        rp[ri, s2] = rp[ri, s2] * ws
                return carry
            lax.fori_loop(0, CH // L, kb, 0)

    def fire_scatters(p):
        for j in range(SUB):
            pltpu.sync_copy(rows[p].at[pl.ds(j * CH, CH)],
                            acc.at[d2[p].at[j]], add=True)

    for p in range(4):
        fire_packed(base + p, p)
    wait_packed(base, 0)
    _idx_from_packed(pk[0], g2[0], gd2[0], d2[0])
    fire_gathers(0)
    fire_packed(base + 4, 0)

    def body(t, carry):
        for u in range(4):
            s = base + 4 * t + u
            pn = (u + 1) % 4
            sn = s + 1

            @pl.when(sn < last)
            def _():
                wait_packed(sn, pn)
                _idx_from_packed(pk[pn], g2[pn], gd2[pn], d2[pn])
                fire_gathers(pn)

                @pl.when(sn + 4 < last)
                def _():
                    fire_packed(sn + 4, pn)

            drain_gathers(u)
            scale(u)
            fire_scatters(u)
        return carry

    lax.fori_loop(0, SUPW // 4, body, 0)
    plsc.subcore_barrier()

    @pl.when(cid == 0)
    def _():
        pltpu.sync_copy(acc.at[pl.ds(sid * ACC_T, ACC_T)],
                        p0_hbm.at[pl.ds(sid * ACC_T, ACC_T)])

    @pl.when(cid == 1)
    def _():
        pltpu.sync_copy(acc.at[pl.ds(sid * ACC_T, ACC_T)],
                        p1_hbm.at[pl.ds(sid * ACC_T, ACC_T)])


def _sc_edge(srcp, dstp, etp, invc, y, zeros2):
    return pl.kernel(
        _sc_edge_body,
        out_type=[jax.ShapeDtypeStruct((NACC, H), jnp.float32),
                  jax.ShapeDtypeStruct((NACC, H), jnp.float32)],
        mesh=_MESH,
        compiler_params=_SC_PARAMS,
        scratch_types=(
            [pltpu.VMEM((3, SED), jnp.int32)] * 4
            + [pltpu.VMEM((SUB, CH), jnp.int32)] * 12
            + [pltpu.VMEM((SUB, CH), jnp.float32)] * 4
            + [pltpu.VMEM((SED, H), jnp.float32)] * 4
            + [pltpu.VMEM_SHARED((NACC, H), jnp.float32)]
            + [pltpu.SemaphoreType.DMA] * 8
        ),
    )(srcp, dstp, etp, invc, y, zeros2)


def kernel(feature, edge_index, edge_type, Wd, bd, Wt, bt, Wn, bn, Wc, bc,
           Wi, bi, W_rel, W_root, b_r, Wo1, bo1, Wo2, bo2):
    f32 = jnp.float32
    i32 = jnp.int32
    wenc = jnp.zeros((FEAT, H), f32)
    wenc = wenc.at[46:814, 0:16].set(Wd)
    wenc = wenc.at[814:1582, 16:32].set(Wt)
    wenc = wenc.at[12:46, 32:48].set(Wn)
    wenc = wenc.at[0:12, 48:64].set(Wc)
    benc = jnp.concatenate([bd, bt, bn, bc]).reshape(1, H)
    bi2 = bi.reshape(1, H)
    br2 = b_r.reshape(1, H)
    bo12 = bo1.reshape(1, H)
    wrelf = jnp.transpose(W_rel, (1, 0, 2)).reshape(H, R * H)
    wo2p = jnp.zeros((H, 128), f32).at[:, :2].set(Wo2)
    bo2p = jnp.zeros((1, 128), f32).at[0, :2].set(bo2)

    srcp = jnp.concatenate([edge_index[0], jnp.zeros((PAD,), i32)])
    dstp = jnp.concatenate([edge_index[1], jnp.full((PAD,), N, i32)])
    etp = jnp.concatenate([edge_type, jnp.zeros((PAD,), i32)])
    zeros1 = jnp.zeros((NRT,), f32)
    zeros2 = jnp.zeros((NACC, H), f32)

    y1, root1 = _tc_encoder(feature.T, wenc, benc, Wi, bi2, wrelf, W_root,
                            br2)

    c0, c1 = _sc_counts(srcp, dstp, etp, zeros1)
    invc = _sc_inv(c0, c1)

    p0, p1 = _sc_edge(srcp, dstp, etp, invc, y1.reshape(NR, H), zeros2)

    y2, root2 = _tc_mid(p0[:N], p1[:N], root1, wrelf, W_root, br2)

    q0, q1 = _sc_edge(srcp, dstp, etp, invc, y2.reshape(NR, H), zeros2)

    out = _tc_out(q0[:N], q1[:N], root2, Wo1, bo12, wo2p, bo2p)
    return out[:, :2]


# speedup vs baseline: 1.1256x; 1.1256x over previous
import jax
import jax.numpy as jnp
from jax import lax
from jax.experimental import pallas as pl
from jax.experimental.pallas import tpu as pltpu
from jax.experimental.pallas import tpu_sc as plsc

N = 10000
E = 640000
FEAT = 1582
H = 64
R = 12
NR = N * R

NC, NS, L = 2, 16, 16
NW = NC * NS
CH = 128
SUB = 2
SED = SUB * CH
SUPW = 80
EPAD = NW * SUPW * SED
PAD = EPAD - E
NCHUNK = EPAD // CH
NRT = 120320
NRT_W = NRT // NW
NACC = N + 16
ACC_T = NACC // NS
CNT_T = NRT // NS

_MESH = plsc.VectorSubcoreMesh(
    core_axis_name="c", subcore_axis_name="s", num_cores=NC, num_subcores=NS)
_SC_PARAMS = pltpu.CompilerParams(use_tc_tiling_on_sc=False)


def _leaky(x):
    return jnp.where(x > 0, x, 0.01 * x)


def _mm(a, b):
    return lax.dot_general(a, b, (((1,), (0,)), ((), ())),
                           preferred_element_type=jnp.float32)


def _mmT(at, b):
    return lax.dot_general(at, b, (((0,), (0,)), ((), ())),
                           preferred_element_type=jnp.float32)


_NB = 10
_BN = N // _NB


def _enc_body(ft_ref, wenc_ref, benc_ref, wi_ref, bi_ref, wrelf_ref,
              wroot_ref, br_ref, y_ref, root_ref):
    a = _leaky(_mmT(ft_ref[...], wenc_ref[...]) + benc_ref[...])
    x = _leaky(_mm(a, wi_ref[...]) + bi_ref[...])
    y_ref[...] = _mm(x, wrelf_ref[...])
    root_ref[...] = _mm(x, wroot_ref[...]) + br_ref[...]


def _mid_body(p0_ref, p1_ref, root_ref, wrelf_ref, wroot_ref, br_ref,
              y_ref, root2_ref):
    x = p0_ref[...] + p1_ref[...] + root_ref[...]
    y_ref[...] = _mm(x, wrelf_ref[...])
    root2_ref[...] = _mm(x, wroot_ref[...]) + br_ref[...]


def _out_body(p0_ref, p1_ref, root_ref, wo1_ref, bo1_ref, wo2_ref, bo2_ref,
              o_ref):
    x = p0_ref[...] + p1_ref[...] + root_ref[...]
    h = _leaky(_mm(x, wo1_ref[...]) + bo1_ref[...])
    o_ref[...] = _mm(h, wo2_ref[...]) + bo2_ref[...]


def _full(shape):
    return pl.BlockSpec(shape, lambda i: tuple(0 for _ in shape))


def _rows(cols):
    return pl.BlockSpec((_BN, cols), lambda i: (i, 0))


_BNE = 1024


def _tc_encoder(ft, wenc, benc, wi, bi, wrelf, wroot, br):
    return pl.pallas_call(
        _enc_body,
        grid=(pl.cdiv(N, _BNE),),
        in_specs=[pl.BlockSpec((FEAT, _BNE), lambda i: (0, i)),
                  _full((FEAT, H)), _full((1, H)),
                  _full((H, H)), _full((1, H)), _full((H, R * H)),
                  _full((H, H)), _full((1, H))],
        out_specs=[pl.BlockSpec((_BNE, R * H), lambda i: (i, 0)),
                   pl.BlockSpec((_BNE, H), lambda i: (i, 0))],
        out_shape=[jax.ShapeDtypeStruct((N, R * H), jnp.float32),
                   jax.ShapeDtypeStruct((N, H), jnp.float32)],
    )(ft, wenc, benc, wi, bi, wrelf, wroot, br)


def _tc_mid(p0, p1, root, wrelf, wroot, br):
    return pl.pallas_call(
        _mid_body,
        grid=(_NB,),
        in_specs=[_rows(H), _rows(H), _rows(H), _full((H, R * H)),
                  _full((H, H)), _full((1, H))],
        out_specs=[_rows(R * H), _rows(H)],
        out_shape=[jax.ShapeDtypeStruct((N, R * H), jnp.float32),
                   jax.ShapeDtypeStruct((N, H), jnp.float32)],
    )(p0, p1, root, wrelf, wroot, br)


def _tc_out(p0, p1, root, wo1, bo1, wo2p, bo2p):
    return pl.pallas_call(
        _out_body,
        grid=(_NB,),
        in_specs=[_rows(H), _rows(H), _rows(H), _full((H, H)),
                  _full((1, H)), _full((H, 128)), _full((1, 128))],
        out_specs=[_rows(128)],
        out_shape=[jax.ShapeDtypeStruct((N, 128), jnp.float32)],
    )(p0, p1, root, wo1, bo1, wo2p, bo2p)[0]


def _wid():
    return lax.axis_index("s") * NC + lax.axis_index("c")


def _idx_from_packed(pk, g2, gd2, d2):
    for j in range(SUB):
        for k in range(CH // L):
            f = pl.ds(j * CH + k * L, L)
            s = pl.ds(k * L, L)
            sv = pk[0, f]
            dv = pk[1, f]
            ev = pk[2, f]
            g2[j, s] = sv * R + ev
            gd2[j, s] = dv * R + ev
            d2[j, s] = dv


def _sc_counts_body(src_hbm, dst_hbm, et_hbm, zeros1_hbm, c0_hbm, c1_hbm, *sc):
    pk = sc[0:2]
    ci = sc[2:4]
    gdum = sc[4:6]
    ddum = sc[6:8]
    ones_v = sc[8]
    acc = sc[9]
    psem = sc[10:12]
    cid = lax.axis_index("c")
    sid = lax.axis_index("s")
    wid = _wid()
    base = wid * SUPW

    pltpu.sync_copy(zeros1_hbm.at[pl.ds(sid * CNT_T, CNT_T)],
                    acc.at[pl.ds(sid * CNT_T, CNT_T)])
    for k in range(CH // L):
        ones_v[pl.ds(k * L, L)] = jnp.full((L,), 1.0, jnp.float32)
    plsc.subcore_barrier()

    def fire_packed(s_, p):
        pltpu.async_copy(src_hbm.at[pl.ds(s_ * SED, SED)], pk[p].at[0],
                         psem[p])
        pltpu.async_copy(dst_hbm.at[pl.ds(s_ * SED, SED)], pk[p].at[1],
                         psem[p])
        pltpu.async_copy(et_hbm.at[pl.ds(s_ * SED, SED)], pk[p].at[2],
                         psem[p])

    def wait_packed(s_, p):
        pltpu.make_async_copy(src_hbm.at[pl.ds(s_ * SED, SED)], pk[p].at[0],
                              psem[p]).wait()
        pltpu.make_async_copy(dst_hbm.at[pl.ds(s_ * SED, SED)], pk[p].at[1],
                              psem[p]).wait()
        pltpu.make_async_copy(et_hbm.at[pl.ds(s_ * SED, SED)], pk[p].at[2],
                              psem[p]).wait()

    for p in range(2):
        fire_packed(base + p, p)

    def body(t, carry):
        for u in range(2):
            s = base + 2 * t + u
            wait_packed(s, u)
            _idx_from_packed(pk[u], gdum[u], ci[u], ddum[u])

            @pl.when(s + 2 < base + SUPW)
            def _():
                fire_packed(s + 2, u)
            for j in range(SUB):
                pltpu.sync_copy(ones_v, acc.at[ci[u].at[j]], add=True)
        return carry

    lax.fori_loop(0, SUPW // 2, body, 0)
    plsc.subcore_barrier()

    @pl.when(cid == 0)
    def _():
        pltpu.sync_copy(acc.at[pl.ds(sid * CNT_T, CNT_T)],
                        c0_hbm.at[pl.ds(sid * CNT_T, CNT_T)])

    @pl.when(cid == 1)
    def _():
        pltpu.sync_copy(acc.at[pl.ds(sid * CNT_T, CNT_T)],
                        c1_hbm.at[pl.ds(sid * CNT_T, CNT_T)])


def _sc_counts(srcp, dstp, etp, zeros1):
    return pl.kernel(
        _sc_counts_body,
        out_type=[jax.ShapeDtypeStruct((NRT,), jnp.float32),
                  jax.ShapeDtypeStruct((NRT,), jnp.float32)],
        mesh=_MESH,
        compiler_params=_SC_PARAMS,
        scratch_types=(
            [pltpu.VMEM((3, SED), jnp.int32)] * 2
            + [pltpu.VMEM((SUB, CH), jnp.int32)] * 6
            + [pltpu.VMEM((CH,), jnp.float32),
               pltpu.VMEM_SHARED((NRT,), jnp.float32)]
            + [pltpu.SemaphoreType.DMA] * 2
        ),
    )(srcp, dstp, etp, zeros1)


def _sc_inv_body(c0_hbm, c1_hbm, invc_hbm, c0_v, c1_v, iv_v, sem):
    wid = _wid()
    off = wid * NRT_W
    pltpu.sync_copy(c0_hbm.at[pl.ds(off, NRT_W)], c0_v)
    pltpu.sync_copy(c1_hbm.at[pl.ds(off, NRT_W)], c1_v)

    def body(k, carry):
        s = pl.ds(k * L, L)
        iv_v[s] = 1.0 / (c0_v[s] + c1_v[s])
        return carry

    lax.fori_loop(0, NRT_W // L, body, 0)
    pltpu.sync_copy(iv_v, invc_hbm.at[pl.ds(off, NRT_W)])


def _sc_inv(c0, c1):
    return pl.kernel(
        _sc_inv_body,
        out_type=[jax.ShapeDtypeStruct((NRT,), jnp.float32)],
        mesh=_MESH,
        compiler_params=_SC_PARAMS,
        scratch_types=[
            pltpu.VMEM((NRT_W,), jnp.float32),
            pltpu.VMEM((NRT_W,), jnp.float32),
            pltpu.VMEM((NRT_W,), jnp.float32),
            pltpu.SemaphoreType.DMA,
        ],
    )(c0, c1)[0]


def _sc_edge_body(src_hbm, dst_hbm, et_hbm, invc_hbm, y_hbm, zeros2_hbm,
                  p0_hbm, p1_hbm, *sc):
    pk = sc[0:4]
    g2 = sc[4:8]
    gd2 = sc[8:12]
    d2 = sc[12:16]
    w2 = sc[16:20]
    rows = sc[20:24]
    acc = sc[24]
    psem = sc[25:29]
    gsem = sc[29:33]
    cid = lax.axis_index("c")
    sid = lax.axis_index("s")
    wid = _wid()
    base = wid * SUPW
    last = base + SUPW

    pltpu.sync_copy(zeros2_hbm.at[pl.ds(sid * ACC_T, ACC_T)],
                    acc.at[pl.ds(sid * ACC_T, ACC_T)])
    plsc.subcore_barrier()

    def fire_packed(s, p):
        pltpu.async_copy(src_hbm.at[pl.ds(s * SED, SED)], pk[p].at[0],
                         psem[p])
        pltpu.async_copy(dst_hbm.at[pl.ds(s * SED, SED)], pk[p].at[1],
                         psem[p])
        pltpu.async_copy(et_hbm.at[pl.ds(s * SED, SED)], pk[p].at[2],
                         psem[p])

    def wait_packed(s, p):
        pltpu.make_async_copy(src_hbm.at[pl.ds(s * SED, SED)], pk[p].at[0],
                              psem[p]).wait()
        pltpu.make_async_copy(dst_hbm.at[pl.ds(s * SED, SED)], pk[p].at[1],
                              psem[p]).wait()
        pltpu.make_async_copy(et_hbm.at[pl.ds(s * SED, SED)], pk[p].at[2],
                              psem[p]).wait()

    def fire_gathers(p):
        for j in range(SUB):
            pltpu.async_copy(y_hbm.at[g2[p].at[j]],
                             rows[p].at[pl.ds(j * CH, CH)], gsem[p])
            pltpu.async_copy(invc_hbm.at[gd2[p].at[j]], w2[p].at[j], gsem[p])

    def drain_gathers(p):
        for j in range(SUB):
            pltpu.make_async_copy(y_hbm.at[g2[p].at[j]],
                                  rows[p].at[pl.ds(j * CH, CH)],
                                  gsem[p]).wait()
            pltpu.make_async_copy(invc_hbm.at[gd2[p].at[j]], w2[p].at[j],
                                  gsem[p]).wait()

    def scale(p):
        rp, wp = rows[p], w2[p]
        for j in range(SUB):
            def kb(k, carry, _j=j):
                w16 = wp[_j, pl.ds(k * L, L)]
                for l in range(L):
                    ws = w16[l]
                    ri = _j * CH + k * L + l
                    for cc in range(H // L):
                        s2 = pl.ds(cc * L, L)
                        rp[ri, s2] = rp[ri, s2] * ws
                return carry
            lax.fori_loop(0, CH // L, kb, 0)

    def fire_scatters(p):
        for j in range(SUB):
            pltpu.sync_copy(rows[p].at[pl.ds(j * CH, CH)],
                            acc.at[d2[p].at[j]], add=True)

    for p in range(4):
        fire_packed(base + p, p)
    wait_packed(base, 0)
    _idx_from_packed(pk[0], g2[0], gd2[0], d2[0])
    fire_gathers(0)
    fire_packed(base + 4, 0)

    def body(t, carry):
        for u in range(4):
            s = base + 4 * t + u
            pn = (u + 1) % 4
            sn = s + 1

            @pl.when(sn < last)
            def _():
                wait_packed(sn, pn)
                _idx_from_packed(pk[pn], g2[pn], gd2[pn], d2[pn])
                fire_gathers(pn)

                @pl.when(sn + 4 < last)
                def _():
                    fire_packed(sn + 4, pn)

            drain_gathers(u)
            scale(u)
            fire_scatters(u)
        return carry

    lax.fori_loop(0, SUPW // 4, body, 0)
    plsc.subcore_barrier()

    @pl.when(cid == 0)
    def _():
        pltpu.sync_copy(acc.at[pl.ds(sid * ACC_T, ACC_T)],
                        p0_hbm.at[pl.ds(sid * ACC_T, ACC_T)])

    @pl.when(cid == 1)
    def _():
        pltpu.sync_copy(acc.at[pl.ds(sid * ACC_T, ACC_T)],
                        p1_hbm.at[pl.ds(sid * ACC_T, ACC_T)])


def _sc_edge(srcp, dstp, etp, invc, y, zeros2):
    return pl.kernel(
        _sc_edge_body,
        out_type=[jax.ShapeDtypeStruct((NACC, H), jnp.float32),
                  jax.ShapeDtypeStruct((NACC, H), jnp.float32)],
        mesh=_MESH,
        compiler_params=_SC_PARAMS,
        scratch_types=(
            [pltpu.VMEM((3, SED), jnp.int32)] * 4
            + [pltpu.VMEM((SUB, CH), jnp.int32)] * 12
            + [pltpu.VMEM((SUB, CH), jnp.float32)] * 4
            + [pltpu.VMEM((SED, H), jnp.float32)] * 4
            + [pltpu.VMEM_SHARED((NACC, H), jnp.float32)]
            + [pltpu.SemaphoreType.DMA] * 8
        ),
    )(srcp, dstp, etp, invc, y, zeros2)


def kernel(feature, edge_index, edge_type, Wd, bd, Wt, bt, Wn, bn, Wc, bc,
           Wi, bi, W_rel, W_root, b_r, Wo1, bo1, Wo2, bo2):
    f32 = jnp.float32
    i32 = jnp.int32
    wenc = jnp.zeros((FEAT, H), f32)
    wenc = wenc.at[46:814, 0:16].set(Wd)
    wenc = wenc.at[814:1582, 16:32].set(Wt)
    wenc = wenc.at[12:46, 32:48].set(Wn)
    wenc = wenc.at[0:12, 48:64].set(Wc)
    benc = jnp.concatenate([bd, bt, bn, bc]).reshape(1, H)
    bi2 = bi.reshape(1, H)
    br2 = b_r.reshape(1, H)
    bo12 = bo1.reshape(1, H)
    wrelf = jnp.transpose(W_rel, (1, 0, 2)).reshape(H, R * H)
    wo2p = jnp.zeros((H, 128), f32).at[:, :2].set(Wo2)
    bo2p = jnp.zeros((1, 128), f32).at[0, :2].set(bo2)

    srcp = jnp.concatenate([edge_index[0], jnp.zeros((PAD,), i32)])
    dstp = jnp.concatenate([edge_index[1], jnp.full((PAD,), N, i32)])
    etp = jnp.concatenate([edge_type, jnp.zeros((PAD,), i32)])
    zeros1 = jnp.zeros((NRT,), f32)
    zeros2 = jnp.zeros((NACC, H), f32)

    y1, root1 = _tc_encoder(feature.T, wenc, benc, Wi, bi2, wrelf, W_root,
                            br2)

    c0, c1 = _sc_counts(srcp, dstp, etp, zeros1)
    invc = _sc_inv(c0, c1)

    p0, p1 = _sc_edge(srcp, dstp, etp, invc, y1.reshape(NR, H), zeros2)

    y2, root2 = _tc_mid(p0[:N], p1[:N], root1, wrelf, W_root, br2)

    q0, q1 = _sc_edge(srcp, dstp, etp, invc, y2.reshape(NR, H), zeros2)

    out = _tc_out(q0[:N], q1[:N], root2, Wo1, bo12, wo2p, bo2p)
    return out[:, :2]


# speedup vs baseline: 1.2283x; 1.0912x over previous
import jax
import jax.numpy as jnp
from jax import lax
from jax.experimental import pallas as pl
from jax.experimental.pallas import tpu as pltpu
from jax.experimental.pallas import tpu_sc as plsc

N = 10000
E = 640000
FEAT = 1582
H = 64
R = 12
NR = N * R

NC, NS, L = 2, 16, 16
NW = NC * NS
CH = 128
SUB = 2
SED = SUB * CH
SUPW = 80
EPAD = NW * SUPW * SED
PAD = EPAD - E
NCHUNK = EPAD // CH
NRT = 120320
NRT_W = NRT // NW
NACC = N + 16
ACC_T = NACC // NS
CNT_T = NRT // NS
SUP0 = 120
SUP1 = 160 - SUP0

_MESH = plsc.VectorSubcoreMesh(
    core_axis_name="c", subcore_axis_name="s", num_cores=NC, num_subcores=NS)
_SC_PARAMS = pltpu.CompilerParams(use_tc_tiling_on_sc=False)


def _leaky(x):
    return jnp.where(x > 0, x, 0.01 * x)


def _mm(a, b):
    return lax.dot_general(a, b, (((1,), (0,)), ((), ())),
                           preferred_element_type=jnp.float32)


def _mmT(at, b):
    return lax.dot_general(at, b, (((0,), (0,)), ((), ())),
                           preferred_element_type=jnp.float32)


_NB = 10
_BN = N // _NB


def _enc_body(ft_ref, wenc_ref, benc_ref, wi_ref, bi_ref, wrelf_ref,
              wroot_ref, br_ref, y_ref, root_ref):
    a = _leaky(_mmT(ft_ref[...], wenc_ref[...]) + benc_ref[...])
    x = _leaky(_mm(a, wi_ref[...]) + bi_ref[...])
    y_ref[...] = _mm(x, wrelf_ref[...])
    root_ref[...] = _mm(x, wroot_ref[...]) + br_ref[...]


def _mid_body(p0_ref, p1_ref, root_ref, wrelf_ref, wroot_ref, br_ref,
              y_ref, root2_ref):
    x = p0_ref[...] + p1_ref[...] + root_ref[...]
    y_ref[...] = _mm(x, wrelf_ref[...])
    root2_ref[...] = _mm(x, wroot_ref[...]) + br_ref[...]


def _out_body(p0_ref, p1_ref, root_ref, wo1_ref, bo1_ref, wo2_ref, bo2_ref,
              o_ref):
    x = p0_ref[...] + p1_ref[...] + root_ref[...]
    h = _leaky(_mm(x, wo1_ref[...]) + bo1_ref[...])
    o_ref[...] = _mm(h, wo2_ref[...]) + bo2_ref[...]


def _full(shape):
    return pl.BlockSpec(shape, lambda i: tuple(0 for _ in shape))


def _rows(cols):
    return pl.BlockSpec((_BN, cols), lambda i: (i, 0))


_BNE = 1024


def _tc_encoder(ft, wenc, benc, wi, bi, wrelf, wroot, br):
    return pl.pallas_call(
        _enc_body,
        grid=(pl.cdiv(N, _BNE),),
        in_specs=[pl.BlockSpec((FEAT, _BNE), lambda i: (0, i)),
                  _full((FEAT, H)), _full((1, H)),
                  _full((H, H)), _full((1, H)), _full((H, R * H)),
                  _full((H, H)), _full((1, H))],
        out_specs=[pl.BlockSpec((_BNE, R * H), lambda i: (i, 0)),
                   pl.BlockSpec((_BNE, H), lambda i: (i, 0))],
        out_shape=[jax.ShapeDtypeStruct((N, R * H), jnp.float32),
                   jax.ShapeDtypeStruct((N, H), jnp.float32)],
    )(ft, wenc, benc, wi, bi, wrelf, wroot, br)


def _tc_mid(p0, p1, root, wrelf, wroot, br):
    return pl.pallas_call(
        _mid_body,
        grid=(_NB,),
        in_specs=[_rows(H), _rows(H), _rows(H), _full((H, R * H)),
                  _full((H, H)), _full((1, H))],
        out_specs=[_rows(R * H), _rows(H)],
        out_shape=[jax.ShapeDtypeStruct((N, R * H), jnp.float32),
                   jax.ShapeDtypeStruct((N, H), jnp.float32)],
    )(p0, p1, root, wrelf, wroot, br)


def _tc_out(p0, p1, root, wo1, bo1, wo2p, bo2p):
    return pl.pallas_call(
        _out_body,
        grid=(_NB,),
        in_specs=[_rows(H), _rows(H), _rows(H), _full((H, H)),
                  _full((1, H)), _full((H, 128)), _full((1, 128))],
        out_specs=[_rows(128)],
        out_shape=[jax.ShapeDtypeStruct((N, 128), jnp.float32)],
    )(p0, p1, root, wo1, bo1, wo2p, bo2p)[0]


def _wid():
    return lax.axis_index("s") * NC + lax.axis_index("c")


def _idx_from_packed(pk, g2, gd2, d2):
    for j in range(SUB):
        for k in range(CH // L):
            f = pl.ds(j * CH + k * L, L)
            s = pl.ds(k * L, L)
            sv = pk[0, f]
            dv = pk[1, f]
            ev = pk[2, f]
            g2[j, s] = sv * R + ev
            gd2[j, s] = dv * R + ev
            d2[j, s] = dv


def _sc_counts_body(src_hbm, dst_hbm, et_hbm, zeros1_hbm, c0_hbm, c1_hbm, *sc):
    pk = sc[0:2]
    ci = sc[2:4]
    gdum = sc[4:6]
    ddum = sc[6:8]
    ones_v = sc[8]
    acc = sc[9]
    psem = sc[10:12]
    cid = lax.axis_index("c")
    sid = lax.axis_index("s")
    wid = _wid()
    base = wid * SUPW

    pltpu.sync_copy(zeros1_hbm.at[pl.ds(sid * CNT_T, CNT_T)],
                    acc.at[pl.ds(sid * CNT_T, CNT_T)])
    for k in range(CH // L):
        ones_v[pl.ds(k * L, L)] = jnp.full((L,), 1.0, jnp.float32)
    plsc.subcore_barrier()

    def fire_packed(s_, p):
        pltpu.async_copy(src_hbm.at[pl.ds(s_ * SED, SED)], pk[p].at[0],
                         psem[p])
        pltpu.async_copy(dst_hbm.at[pl.ds(s_ * SED, SED)], pk[p].at[1],
                         psem[p])
        pltpu.async_copy(et_hbm.at[pl.ds(s_ * SED, SED)], pk[p].at[2],
                         psem[p])

    def wait_packed(s_, p):
        pltpu.make_async_copy(src_hbm.at[pl.ds(s_ * SED, SED)], pk[p].at[0],
                              psem[p]).wait()
        pltpu.make_async_copy(dst_hbm.at[pl.ds(s_ * SED, SED)], pk[p].at[1],
                              psem[p]).wait()
        pltpu.make_async_copy(et_hbm.at[pl.ds(s_ * SED, SED)], pk[p].at[2],
                              psem[p]).wait()

    for p in range(2):
        fire_packed(base + p, p)

    def body(t, carry):
        for u in range(2):
            s = base + 2 * t + u
            wait_packed(s, u)
            _idx_from_packed(pk[u], gdum[u], ci[u], ddum[u])

            @pl.when(s + 2 < base + SUPW)
            def _():
                fire_packed(s + 2, u)
            for j in range(SUB):
                pltpu.sync_copy(ones_v, acc.at[ci[u].at[j]], add=True)
        return carry

    lax.fori_loop(0, SUPW // 2, body, 0)
    plsc.subcore_barrier()

    @pl.when(cid == 0)
    def _():
        pltpu.sync_copy(acc.at[pl.ds(sid * CNT_T, CNT_T)],
                        c0_hbm.at[pl.ds(sid * CNT_T, CNT_T)])

    @pl.when(cid == 1)
    def _():
        pltpu.sync_copy(acc.at[pl.ds(sid * CNT_T, CNT_T)],
                        c1_hbm.at[pl.ds(sid * CNT_T, CNT_T)])


def _sc_counts(srcp, dstp, etp, zeros1):
    return pl.kernel(
        _sc_counts_body,
        out_type=[jax.ShapeDtypeStruct((NRT,), jnp.float32),
                  jax.ShapeDtypeStruct((NRT,), jnp.float32)],
        mesh=_MESH,
        compiler_params=_SC_PARAMS,
        scratch_types=(
            [pltpu.VMEM((3, SED), jnp.int32)] * 2
            + [pltpu.VMEM((SUB, CH), jnp.int32)] * 6
            + [pltpu.VMEM((CH,), jnp.float32),
               pltpu.VMEM_SHARED((NRT,), jnp.float32)]
            + [pltpu.SemaphoreType.DMA] * 2
        ),
    )(srcp, dstp, etp, zeros1)


def _sc_inv_body(c0_hbm, c1_hbm, invc_hbm, c0_v, c1_v, iv_v, sem):
    wid = _wid()
    off = wid * NRT_W
    pltpu.sync_copy(c0_hbm.at[pl.ds(off, NRT_W)], c0_v)
    pltpu.sync_copy(c1_hbm.at[pl.ds(off, NRT_W)], c1_v)

    def body(k, carry):
        s = pl.ds(k * L, L)
        iv_v[s] = 1.0 / (c0_v[s] + c1_v[s])
        return carry

    lax.fori_loop(0, NRT_W // L, body, 0)
    pltpu.sync_copy(iv_v, invc_hbm.at[pl.ds(off, NRT_W)])


def _sc_inv(c0, c1):
    return pl.kernel(
        _sc_inv_body,
        out_type=[jax.ShapeDtypeStruct((NRT,), jnp.float32)],
        mesh=_MESH,
        compiler_params=_SC_PARAMS,
        scratch_types=[
            pltpu.VMEM((NRT_W,), jnp.float32),
            pltpu.VMEM((NRT_W,), jnp.float32),
            pltpu.VMEM((NRT_W,), jnp.float32),
            pltpu.SemaphoreType.DMA,
        ],
    )(c0, c1)[0]


def _sc_edge_body(src_hbm, dst_hbm, et_hbm, invc_hbm, y_hbm, zeros2_hbm,
                  p0_hbm, p1_hbm, *sc):
    pk = sc[0:4]
    g2 = sc[4:8]
    gd2 = sc[8:12]
    d2 = sc[12:16]
    w2 = sc[16:20]
    rows = sc[20:24]
    acc = sc[24]
    psem = sc[25:29]
    gsem = sc[29:33]
    cid = lax.axis_index("c")
    sid = lax.axis_index("s")
    base = jnp.where(cid == 0, sid * SUP0, NS * SUP0 + sid * SUP1)
    nsup = jnp.where(cid == 0, SUP0, SUP1)
    last = base + nsup

    pltpu.sync_copy(zeros2_hbm.at[pl.ds(sid * ACC_T, ACC_T)],
                    acc.at[pl.ds(sid * ACC_T, ACC_T)])
    plsc.subcore_barrier()

    def fire_packed(s, p):
        pltpu.async_copy(src_hbm.at[pl.ds(s * SED, SED)], pk[p].at[0],
                         psem[p])
        pltpu.async_copy(dst_hbm.at[pl.ds(s * SED, SED)], pk[p].at[1],
                         psem[p])
        pltpu.async_copy(et_hbm.at[pl.ds(s * SED, SED)], pk[p].at[2],
                         psem[p])

    def wait_packed(s, p):
        pltpu.make_async_copy(src_hbm.at[pl.ds(s * SED, SED)], pk[p].at[0],
                              psem[p]).wait()
        pltpu.make_async_copy(dst_hbm.at[pl.ds(s * SED, SED)], pk[p].at[1],
                              psem[p]).wait()
        pltpu.make_async_copy(et_hbm.at[pl.ds(s * SED, SED)], pk[p].at[2],
                              psem[p]).wait()

    def fire_gathers(p):
        for j in range(SUB):
            pltpu.async_copy(y_hbm.at[g2[p].at[j]],
                             rows[p].at[pl.ds(j * CH, CH)], gsem[p])
            pltpu.async_copy(invc_hbm.at[gd2[p].at[j]], w2[p].at[j], gsem[p])

    def drain_gathers(p):
        for j in range(SUB):
            pltpu.make_async_copy(y_hbm.at[g2[p].at[j]],
                                  rows[p].at[pl.ds(j * CH, CH)],
                                  gsem[p]).wait()
            pltpu.make_async_copy(invc_hbm.at[gd2[p].at[j]], w2[p].at[j],
                                  gsem[p]).wait()

    def scale(p):
        rp, wp = rows[p], w2[p]
        for j in range(SUB):
            def kb(k, carry, _j=j):
                w16 = wp[_j, pl.ds(k * L, L)]
                for l in range(L):
                    ws = w16[l]
                    ri = _j * CH + k * L + l
                    for cc in range(H // L):
                        s2 = pl.ds(cc * L, L)
                        rp[ri, s2] = rp[ri, s2] * ws
                return carry
            lax.fori_loop(0, CH // L, kb, 0)

    def fire_scatters(p):
        for j in range(SUB):
            pltpu.sync_copy(rows[p].at[pl.ds(j * CH, CH)],
                            acc.at[d2[p].at[j]], add=True)

    for p in range(4):
        fire_packed(base + p, p)
    wait_packed(base, 0)
    _idx_from_packed(pk[0], g2[0], gd2[0], d2[0])
    fire_gathers(0)
    fire_packed(base + 4, 0)

    def body(t, carry):
        for u in range(4):
            s = base + 4 * t + u
            pn = (u + 1) % 4
            sn = s + 1

            @pl.when(sn < last)
            def _():
                wait_packed(sn, pn)
                _idx_from_packed(pk[pn], g2[pn], gd2[pn], d2[pn])
                fire_gathers(pn)

                @pl.when(sn + 4 < last)
                def _():
                    fire_packed(sn + 4, pn)

            drain_gathers(u)
            scale(u)
            fire_scatters(u)
        return carry

    lax.fori_loop(0, nsup // 4, body, 0)
    plsc.subcore_barrier()

    @pl.when(cid == 0)
    def _():
        pltpu.sync_copy(acc.at[pl.ds(sid * ACC_T, ACC_T)],
                        p0_hbm.at[pl.ds(sid * ACC_T, ACC_T)])

    @pl.when(cid == 1)
    def _():
        pltpu.sync_copy(acc.at[pl.ds(sid * ACC_T, ACC_T)],
                        p1_hbm.at[pl.ds(sid * ACC_T, ACC_T)])


def _sc_edge(srcp, dstp, etp, invc, y, zeros2):
    return pl.kernel(
        _sc_edge_body,
        out_type=[jax.ShapeDtypeStruct((NACC, H), jnp.float32),
                  jax.ShapeDtypeStruct((NACC, H), jnp.float32)],
        mesh=_MESH,
        compiler_params=_SC_PARAMS,
        scratch_types=(
            [pltpu.VMEM((3, SED), jnp.int32)] * 4
            + [pltpu.VMEM((SUB, CH), jnp.int32)] * 12
            + [pltpu.VMEM((SUB, CH), jnp.float32)] * 4
            + [pltpu.VMEM((SED, H), jnp.float32)] * 4
            + [pltpu.VMEM_SHARED((NACC, H), jnp.float32)]
            + [pltpu.SemaphoreType.DMA] * 8
        ),
    )(srcp, dstp, etp, invc, y, zeros2)


def kernel(feature, edge_index, edge_type, Wd, bd, Wt, bt, Wn, bn, Wc, bc,
           Wi, bi, W_rel, W_root, b_r, Wo1, bo1, Wo2, bo2):
    f32 = jnp.float32
    i32 = jnp.int32
    wenc = jnp.zeros((FEAT, H), f32)
    wenc = wenc.at[46:814, 0:16].set(Wd)
    wenc = wenc.at[814:1582, 16:32].set(Wt)
    wenc = wenc.at[12:46, 32:48].set(Wn)
    wenc = wenc.at[0:12, 48:64].set(Wc)
    benc = jnp.concatenate([bd, bt, bn, bc]).reshape(1, H)
    bi2 = bi.reshape(1, H)
    br2 = b_r.reshape(1, H)
    bo12 = bo1.reshape(1, H)
    wrelf = jnp.transpose(W_rel, (1, 0, 2)).reshape(H, R * H)
    wo2p = jnp.zeros((H, 128), f32).at[:, :2].set(Wo2)
    bo2p = jnp.zeros((1, 128), f32).at[0, :2].set(bo2)

    srcp = jnp.concatenate([edge_index[0], jnp.zeros((PAD,), i32)])
    dstp = jnp.concatenate([edge_index[1], jnp.full((PAD,), N, i32)])
    etp = jnp.concatenate([edge_type, jnp.zeros((PAD,), i32)])
    zeros1 = jnp.zeros((NRT,), f32)
    zeros2 = jnp.zeros((NACC, H), f32)

    y1, root1 = _tc_encoder(feature.T, wenc, benc, Wi, bi2, wrelf, W_root,
                            br2)

    c0, c1 = _sc_counts(srcp, dstp, etp, zeros1)
    invc = _sc_inv(c0, c1)

    p0, p1 = _sc_edge(srcp, dstp, etp, invc, y1.reshape(NR, H), zeros2)

    y2, root2 = _tc_mid(p0[:N], p1[:N], root1, wrelf, W_root, br2)

    q0, q1 = _sc_edge(srcp, dstp, etp, invc, y2.reshape(NR, H), zeros2)

    out = _tc_out(q0[:N], q1[:N], root2, Wo1, bo12, wo2p, bo2p)
    return out[:, :2]


# speedup vs baseline: 1.3147x; 1.0703x over previous
import jax
import jax.numpy as jnp
from jax import lax
from jax.experimental import pallas as pl
from jax.experimental.pallas import tpu as pltpu
from jax.experimental.pallas import tpu_sc as plsc

N = 10000
E = 640000
FEAT = 1582
H = 64
R = 12
NR = N * R

NC, NS, L = 2, 16, 16
NW = NC * NS
CH = 128
SUB = 2
SED = SUB * CH
SUPW = 80
EPAD = NW * SUPW * SED
PAD = EPAD - E
NCHUNK = EPAD // CH
NRT = 120320
NRT_W = NRT // NW
NACC = N + 16
ACC_T = NACC // NS
CNT_T = NRT // NS
SUP0 = 140
SUP1 = 160 - SUP0

_MESH = plsc.VectorSubcoreMesh(
    core_axis_name="c", subcore_axis_name="s", num_cores=NC, num_subcores=NS)
_SC_PARAMS = pltpu.CompilerParams(use_tc_tiling_on_sc=False)


def _leaky(x):
    return jnp.where(x > 0, x, 0.01 * x)


def _mm(a, b):
    return lax.dot_general(a, b, (((1,), (0,)), ((), ())),
                           preferred_element_type=jnp.float32)


def _mmT(at, b):
    return lax.dot_general(at, b, (((0,), (0,)), ((), ())),
                           preferred_element_type=jnp.float32)


_NB = 10
_BN = N // _NB


def _enc_body(ft_ref, wenc_ref, benc_ref, wi_ref, bi_ref, wrelf_ref,
              wroot_ref, br_ref, y_ref, root_ref):
    a = _leaky(_mmT(ft_ref[...], wenc_ref[...]) + benc_ref[...])
    x = _leaky(_mm(a, wi_ref[...]) + bi_ref[...])
    y_ref[...] = _mm(x, wrelf_ref[...])
    root_ref[...] = _mm(x, wroot_ref[...]) + br_ref[...]


def _mid_body(p0_ref, p1_ref, root_ref, wrelf_ref, wroot_ref, br_ref,
              y_ref, root2_ref):
    x = p0_ref[...] + p1_ref[...] + root_ref[...]
    y_ref[...] = _mm(x, wrelf_ref[...])
    root2_ref[...] = _mm(x, wroot_ref[...]) + br_ref[...]


def _out_body(p0_ref, p1_ref, root_ref, wo1_ref, bo1_ref, wo2_ref, bo2_ref,
              o_ref):
    x = p0_ref[...] + p1_ref[...] + root_ref[...]
    h = _leaky(_mm(x, wo1_ref[...]) + bo1_ref[...])
    o_ref[...] = _mm(h, wo2_ref[...]) + bo2_ref[...]


def _full(shape):
    return pl.BlockSpec(shape, lambda i: tuple(0 for _ in shape))


def _rows(cols):
    return pl.BlockSpec((_BN, cols), lambda i: (i, 0))


_BNE = 1024


def _tc_encoder(ft, wenc, benc, wi, bi, wrelf, wroot, br):
    return pl.pallas_call(
        _enc_body,
        grid=(pl.cdiv(N, _BNE),),
        in_specs=[pl.BlockSpec((FEAT, _BNE), lambda i: (0, i)),
                  _full((FEAT, H)), _full((1, H)),
                  _full((H, H)), _full((1, H)), _full((H, R * H)),
                  _full((H, H)), _full((1, H))],
        out_specs=[pl.BlockSpec((_BNE, R * H), lambda i: (i, 0)),
                   pl.BlockSpec((_BNE, H), lambda i: (i, 0))],
        out_shape=[jax.ShapeDtypeStruct((N, R * H), jnp.float32),
                   jax.ShapeDtypeStruct((N, H), jnp.float32)],
    )(ft, wenc, benc, wi, bi, wrelf, wroot, br)


def _tc_mid(p0, p1, root, wrelf, wroot, br):
    return pl.pallas_call(
        _mid_body,
        grid=(_NB,),
        in_specs=[_rows(H), _rows(H), _rows(H), _full((H, R * H)),
                  _full((H, H)), _full((1, H))],
        out_specs=[_rows(R * H), _rows(H)],
        out_shape=[jax.ShapeDtypeStruct((N, R * H), jnp.float32),
                   jax.ShapeDtypeStruct((N, H), jnp.float32)],
    )(p0, p1, root, wrelf, wroot, br)


def _tc_out(p0, p1, root, wo1, bo1, wo2p, bo2p):
    return pl.pallas_call(
        _out_body,
        grid=(_NB,),
        in_specs=[_rows(H), _rows(H), _rows(H), _full((H, H)),
                  _full((1, H)), _full((H, 128)), _full((1, 128))],
        out_specs=[_rows(128)],
        out_shape=[jax.ShapeDtypeStruct((N, 128), jnp.float32)],
    )(p0, p1, root, wo1, bo1, wo2p, bo2p)[0]


def _wid():
    return lax.axis_index("s") * NC + lax.axis_index("c")


def _idx_from_packed(pk, g2, gd2, d2):
    for j in range(SUB):
        for k in range(CH // L):
            f = pl.ds(j * CH + k * L, L)
            s = pl.ds(k * L, L)
            sv = pk[0, f]
            dv = pk[1, f]
            ev = pk[2, f]
            g2[j, s] = sv * R + ev
            gd2[j, s] = dv * R + ev
            d2[j, s] = dv


def _sc_counts_body(src_hbm, dst_hbm, et_hbm, zeros1_hbm, c0_hbm, c1_hbm, *sc):
    pk = sc[0:2]
    ci = sc[2:4]
    gdum = sc[4:6]
    ddum = sc[6:8]
    ones_v = sc[8]
    acc = sc[9]
    psem = sc[10:12]
    cid = lax.axis_index("c")
    sid = lax.axis_index("s")
    wid = _wid()
    base = wid * SUPW

    pltpu.sync_copy(zeros1_hbm.at[pl.ds(sid * CNT_T, CNT_T)],
                    acc.at[pl.ds(sid * CNT_T, CNT_T)])
    for k in range(CH // L):
        ones_v[pl.ds(k * L, L)] = jnp.full((L,), 1.0, jnp.float32)
    plsc.subcore_barrier()

    def fire_packed(s_, p):
        pltpu.async_copy(src_hbm.at[pl.ds(s_ * SED, SED)], pk[p].at[0],
                         psem[p])
        pltpu.async_copy(dst_hbm.at[pl.ds(s_ * SED, SED)], pk[p].at[1],
                         psem[p])
        pltpu.async_copy(et_hbm.at[pl.ds(s_ * SED, SED)], pk[p].at[2],
                         psem[p])

    def wait_packed(s_, p):
        pltpu.make_async_copy(src_hbm.at[pl.ds(s_ * SED, SED)], pk[p].at[0],
                              psem[p]).wait()
        pltpu.make_async_copy(dst_hbm.at[pl.ds(s_ * SED, SED)], pk[p].at[1],
                              psem[p]).wait()
        pltpu.make_async_copy(et_hbm.at[pl.ds(s_ * SED, SED)], pk[p].at[2],
                              psem[p]).wait()

    for p in range(2):
        fire_packed(base + p, p)

    def body(t, carry):
        for u in range(2):
            s = base + 2 * t + u
            wait_packed(s, u)
            _idx_from_packed(pk[u], gdum[u], ci[u], ddum[u])

            @pl.when(s + 2 < base + SUPW)
            def _():
                fire_packed(s + 2, u)
            for j in range(SUB):
                pltpu.sync_copy(ones_v, acc.at[ci[u].at[j]], add=True)
        return carry

    lax.fori_loop(0, SUPW // 2, body, 0)
    plsc.subcore_barrier()

    @pl.when(cid == 0)
    def _():
        pltpu.sync_copy(acc.at[pl.ds(sid * CNT_T, CNT_T)],
                        c0_hbm.at[pl.ds(sid * CNT_T, CNT_T)])

    @pl.when(cid == 1)
    def _():
        pltpu.sync_copy(acc.at[pl.ds(sid * CNT_T, CNT_T)],
                        c1_hbm.at[pl.ds(sid * CNT_T, CNT_T)])


def _sc_counts(srcp, dstp, etp, zeros1):
    return pl.kernel(
        _sc_counts_body,
        out_type=[jax.ShapeDtypeStruct((NRT,), jnp.float32),
                  jax.ShapeDtypeStruct((NRT,), jnp.float32)],
        mesh=_MESH,
        compiler_params=_SC_PARAMS,
        scratch_types=(
            [pltpu.VMEM((3, SED), jnp.int32)] * 2
            + [pltpu.VMEM((SUB, CH), jnp.int32)] * 6
            + [pltpu.VMEM((CH,), jnp.float32),
               pltpu.VMEM_SHARED((NRT,), jnp.float32)]
            + [pltpu.SemaphoreType.DMA] * 2
        ),
    )(srcp, dstp, etp, zeros1)


def _sc_inv_body(c0_hbm, c1_hbm, invc_hbm, c0_v, c1_v, iv_v, sem):
    wid = _wid()
    off = wid * NRT_W
    pltpu.sync_copy(c0_hbm.at[pl.ds(off, NRT_W)], c0_v)
    pltpu.sync_copy(c1_hbm.at[pl.ds(off, NRT_W)], c1_v)

    def body(k, carry):
        s = pl.ds(k * L, L)
        iv_v[s] = 1.0 / (c0_v[s] + c1_v[s])
        return carry

    lax.fori_loop(0, NRT_W // L, body, 0)
    pltpu.sync_copy(iv_v, invc_hbm.at[pl.ds(off, NRT_W)])


def _sc_inv(c0, c1):
    return pl.kernel(
        _sc_inv_body,
        out_type=[jax.ShapeDtypeStruct((NRT,), jnp.float32)],
        mesh=_MESH,
        compiler_params=_SC_PARAMS,
        scratch_types=[
            pltpu.VMEM((NRT_W,), jnp.float32),
            pltpu.VMEM((NRT_W,), jnp.float32),
            pltpu.VMEM((NRT_W,), jnp.float32),
            pltpu.SemaphoreType.DMA,
        ],
    )(c0, c1)[0]


def _sc_edge_body(src_hbm, dst_hbm, et_hbm, invc_hbm, y_hbm, zeros2_hbm,
                  p0_hbm, p1_hbm, *sc):
    pk = sc[0:4]
    g2 = sc[4:8]
    gd2 = sc[8:12]
    d2 = sc[12:16]
    w2 = sc[16:20]
    rows = sc[20:24]
    acc = sc[24]
    psem = sc[25:29]
    gsem = sc[29:33]
    cid = lax.axis_index("c")
    sid = lax.axis_index("s")
    base = jnp.where(cid == 0, sid * SUP0, NS * SUP0 + sid * SUP1)
    nsup = jnp.where(cid == 0, SUP0, SUP1)
    last = base + nsup

    pltpu.sync_copy(zeros2_hbm.at[pl.ds(sid * ACC_T, ACC_T)],
                    acc.at[pl.ds(sid * ACC_T, ACC_T)])
    plsc.subcore_barrier()

    def fire_packed(s, p):
        pltpu.async_copy(src_hbm.at[pl.ds(s * SED, SED)], pk[p].at[0],
                         psem[p])
        pltpu.async_copy(dst_hbm.at[pl.ds(s * SED, SED)], pk[p].at[1],
                         psem[p])
        pltpu.async_copy(et_hbm.at[pl.ds(s * SED, SED)], pk[p].at[2],
                         psem[p])

    def wait_packed(s, p):
        pltpu.make_async_copy(src_hbm.at[pl.ds(s * SED, SED)], pk[p].at[0],
                              psem[p]).wait()
        pltpu.make_async_copy(dst_hbm.at[pl.ds(s * SED, SED)], pk[p].at[1],
                              psem[p]).wait()
        pltpu.make_async_copy(et_hbm.at[pl.ds(s * SED, SED)], pk[p].at[2],
                              psem[p]).wait()

    def fire_gathers(p):
        for j in range(SUB):
            pltpu.async_copy(y_hbm.at[g2[p].at[j]],
                             rows[p].at[pl.ds(j * CH, CH)], gsem[p])
            pltpu.async_copy(invc_hbm.at[gd2[p].at[j]], w2[p].at[j], gsem[p])

    def drain_gathers(p):
        for j in range(SUB):
            pltpu.make_async_copy(y_hbm.at[g2[p].at[j]],
                                  rows[p].at[pl.ds(j * CH, CH)],
                                  gsem[p]).wait()
            pltpu.make_async_copy(invc_hbm.at[gd2[p].at[j]], w2[p].at[j],
                                  gsem[p]).wait()

    def scale(p):
        rp, wp = rows[p], w2[p]
        for j in range(SUB):
            def kb(k, carry, _j=j):
                w16 = wp[_j, pl.ds(k * L, L)]
                for l in range(L):
                    ws = w16[l]
                    ri = _j * CH + k * L + l
                    for cc in range(H // L):
                        s2 = pl.ds(cc * L, L)
                        rp[ri, s2] = rp[ri, s2] * ws
                return carry
            lax.fori_loop(0, CH // L, kb, 0)

    def fire_scatters(p):
        for j in range(SUB):
            pltpu.sync_copy(rows[p].at[pl.ds(j * CH, CH)],
                            acc.at[d2[p].at[j]], add=True)

    for p in range(4):
        fire_packed(base + p, p)
    wait_packed(base, 0)
    _idx_from_packed(pk[0], g2[0], gd2[0], d2[0])
    fire_gathers(0)
    fire_packed(base + 4, 0)

    def body(t, carry):
        for u in range(4):
            s = base + 4 * t + u
            pn = (u + 1) % 4
            sn = s + 1

            @pl.when(sn < last)
            def _():
                wait_packed(sn, pn)
                _idx_from_packed(pk[pn], g2[pn], gd2[pn], d2[pn])
                fire_gathers(pn)

                @pl.when(sn + 4 < last)
                def _():
                    fire_packed(sn + 4, pn)

            drain_gathers(u)
            scale(u)
            fire_scatters(u)
        return carry

    lax.fori_loop(0, nsup // 4, body, 0)
    plsc.subcore_barrier()

    @pl.when(cid == 0)
    def _():
        pltpu.sync_copy(acc.at[pl.ds(sid * ACC_T, ACC_T)],
                        p0_hbm.at[pl.ds(sid * ACC_T, ACC_T)])

    @pl.when(cid == 1)
    def _():
        pltpu.sync_copy(acc.at[pl.ds(sid * ACC_T, ACC_T)],
                        p1_hbm.at[pl.ds(sid * ACC_T, ACC_T)])


def _sc_edge(srcp, dstp, etp, invc, y, zeros2):
    return pl.kernel(
        _sc_edge_body,
        out_type=[jax.ShapeDtypeStruct((NACC, H), jnp.float32),
                  jax.ShapeDtypeStruct((NACC, H), jnp.float32)],
        mesh=_MESH,
        compiler_params=_SC_PARAMS,
        scratch_types=(
            [pltpu.VMEM((3, SED), jnp.int32)] * 4
            + [pltpu.VMEM((SUB, CH), jnp.int32)] * 12
            + [pltpu.VMEM((SUB, CH), jnp.float32)] * 4
            + [pltpu.VMEM((SED, H), jnp.float32)] * 4
            + [pltpu.VMEM_SHARED((NACC, H), jnp.float32)]
            + [pltpu.SemaphoreType.DMA] * 8
        ),
    )(srcp, dstp, etp, invc, y, zeros2)


def kernel(feature, edge_index, edge_type, Wd, bd, Wt, bt, Wn, bn, Wc, bc,
           Wi, bi, W_rel, W_root, b_r, Wo1, bo1, Wo2, bo2):
    f32 = jnp.float32
    i32 = jnp.int32
    wenc = jnp.zeros((FEAT, H), f32)
    wenc = wenc.at[46:814, 0:16].set(Wd)
    wenc = wenc.at[814:1582, 16:32].set(Wt)
    wenc = wenc.at[12:46, 32:48].set(Wn)
    wenc = wenc.at[0:12, 48:64].set(Wc)
    benc = jnp.concatenate([bd, bt, bn, bc]).reshape(1, H)
    bi2 = bi.reshape(1, H)
    br2 = b_r.reshape(1, H)
    bo12 = bo1.reshape(1, H)
    wrelf = jnp.transpose(W_rel, (1, 0, 2)).reshape(H, R * H)
    wo2p = jnp.zeros((H, 128), f32).at[:, :2].set(Wo2)
    bo2p = jnp.zeros((1, 128), f32).at[0, :2].set(bo2)

    srcp = jnp.concatenate([edge_index[0], jnp.zeros((PAD,), i32)])
    dstp = jnp.concatenate([edge_index[1], jnp.full((PAD,), N, i32)])
    etp = jnp.concatenate([edge_type, jnp.zeros((PAD,), i32)])
    zeros1 = jnp.zeros((NRT,), f32)
    zeros2 = jnp.zeros((NACC, H), f32)

    y1, root1 = _tc_encoder(feature.T, wenc, benc, Wi, bi2, wrelf, W_root,
                            br2)

    c0, c1 = _sc_counts(srcp, dstp, etp, zeros1)
    invc = _sc_inv(c0, c1)

    p0, p1 = _sc_edge(srcp, dstp, etp, invc, y1.reshape(NR, H), zeros2)

    y2, root2 = _tc_mid(p0[:N], p1[:N], root1, wrelf, W_root, br2)

    q0, q1 = _sc_edge(srcp, dstp, etp, invc, y2.reshape(NR, H), zeros2)

    out = _tc_out(q0[:N], q1[:N], root2, Wo1, bo12, wo2p, bo2p)
    return out[:, :2]


# speedup vs baseline: 1.3585x; 1.0333x over previous
import jax
import jax.numpy as jnp
from jax import lax
from jax.experimental import pallas as pl
from jax.experimental.pallas import tpu as pltpu
from jax.experimental.pallas import tpu_sc as plsc

N = 10000
E = 640000
FEAT = 1582
H = 64
R = 12
NR = N * R

NC, NS, L = 2, 16, 16
NW = NC * NS
CH = 128
SUB = 2
SED = SUB * CH
SUPW = 80
EPAD = NW * SUPW * SED
PAD = EPAD - E
NCHUNK = EPAD // CH
NRT = 120320
NRT_W = NRT // NW
NACC = N + 16
ACC_T = NACC // NS
CNT_T = NRT // NS
SUP0 = 152
SUP1 = 160 - SUP0

_MESH = plsc.VectorSubcoreMesh(
    core_axis_name="c", subcore_axis_name="s", num_cores=NC, num_subcores=NS)
_SC_PARAMS = pltpu.CompilerParams(use_tc_tiling_on_sc=False)


def _leaky(x):
    return jnp.where(x > 0, x, 0.01 * x)


def _mm(a, b):
    return lax.dot_general(a, b, (((1,), (0,)), ((), ())),
                           preferred_element_type=jnp.float32)


def _mmT(at, b):
    return lax.dot_general(at, b, (((0,), (0,)), ((), ())),
                           preferred_element_type=jnp.float32)


_NB = 10
_BN = N // _NB


def _enc_body(ft_ref, wenc_ref, benc_ref, wi_ref, bi_ref, wrelf_ref,
              wroot_ref, br_ref, y_ref, root_ref):
    a = _leaky(_mmT(ft_ref[...], wenc_ref[...]) + benc_ref[...])
    x = _leaky(_mm(a, wi_ref[...]) + bi_ref[...])
    y_ref[...] = _mm(x, wrelf_ref[...])
    root_ref[...] = _mm(x, wroot_ref[...]) + br_ref[...]


def _mid_body(p0_ref, p1_ref, root_ref, wrelf_ref, wroot_ref, br_ref,
              y_ref, root2_ref):
    x = p0_ref[...] + p1_ref[...] + root_ref[...]
    y_ref[...] = _mm(x, wrelf_ref[...])
    root2_ref[...] = _mm(x, wroot_ref[...]) + br_ref[...]


def _out_body(p0_ref, p1_ref, root_ref, wo1_ref, bo1_ref, wo2_ref, bo2_ref,
              o_ref):
    x = p0_ref[...] + p1_ref[...] + root_ref[...]
    h = _leaky(_mm(x, wo1_ref[...]) + bo1_ref[...])
    o_ref[...] = _mm(h, wo2_ref[...]) + bo2_ref[...]


def _full(shape):
    return pl.BlockSpec(shape, lambda i: tuple(0 for _ in shape))


def _rows(cols):
    return pl.BlockSpec((_BN, cols), lambda i: (i, 0))


_BNE = 1024


def _tc_encoder(ft, wenc, benc, wi, bi, wrelf, wroot, br):
    return pl.pallas_call(
        _enc_body,
        grid=(pl.cdiv(N, _BNE),),
        in_specs=[pl.BlockSpec((FEAT, _BNE), lambda i: (0, i)),
                  _full((FEAT, H)), _full((1, H)),
                  _full((H, H)), _full((1, H)), _full((H, R * H)),
                  _full((H, H)), _full((1, H))],
        out_specs=[pl.BlockSpec((_BNE, R * H), lambda i: (i, 0)),
                   pl.BlockSpec((_BNE, H), lambda i: (i, 0))],
        out_shape=[jax.ShapeDtypeStruct((N, R * H), jnp.float32),
                   jax.ShapeDtypeStruct((N, H), jnp.float32)],
    )(ft, wenc, benc, wi, bi, wrelf, wroot, br)


def _tc_mid(p0, p1, root, wrelf, wroot, br):
    return pl.pallas_call(
        _mid_body,
        grid=(_NB,),
        in_specs=[_rows(H), _rows(H), _rows(H), _full((H, R * H)),
                  _full((H, H)), _full((1, H))],
        out_specs=[_rows(R * H), _rows(H)],
        out_shape=[jax.ShapeDtypeStruct((N, R * H), jnp.float32),
                   jax.ShapeDtypeStruct((N, H), jnp.float32)],
    )(p0, p1, root, wrelf, wroot, br)


def _tc_out(p0, p1, root, wo1, bo1, wo2p, bo2p):
    return pl.pallas_call(
        _out_body,
        grid=(_NB,),
        in_specs=[_rows(H), _rows(H), _rows(H), _full((H, H)),
                  _full((1, H)), _full((H, 128)), _full((1, 128))],
        out_specs=[_rows(128)],
        out_shape=[jax.ShapeDtypeStruct((N, 128), jnp.float32)],
    )(p0, p1, root, wo1, bo1, wo2p, bo2p)[0]


def _wid():
    return lax.axis_index("s") * NC + lax.axis_index("c")


def _idx_from_packed(pk, g2, gd2, d2):
    for j in range(SUB):
        for k in range(CH // L):
            f = pl.ds(j * CH + k * L, L)
            s = pl.ds(k * L, L)
            sv = pk[0, f]
            dv = pk[1, f]
            ev = pk[2, f]
            g2[j, s] = sv * R + ev
            gd2[j, s] = dv * R + ev
            d2[j, s] = dv


def _sc_counts_body(src_hbm, dst_hbm, et_hbm, zeros1_hbm, c0_hbm, c1_hbm, *sc):
    pk = sc[0:2]
    ci = sc[2:4]
    gdum = sc[4:6]
    ddum = sc[6:8]
    ones_v = sc[8]
    acc = sc[9]
    psem = sc[10:12]
    cid = lax.axis_index("c")
    sid = lax.axis_index("s")
    wid = _wid()
    base = wid * SUPW

    pltpu.sync_copy(zeros1_hbm.at[pl.ds(sid * CNT_T, CNT_T)],
                    acc.at[pl.ds(sid * CNT_T, CNT_T)])
    for k in range(CH // L):
        ones_v[pl.ds(k * L, L)] = jnp.full((L,), 1.0, jnp.float32)
    plsc.subcore_barrier()

    def fire_packed(s_, p):
        pltpu.async_copy(src_hbm.at[pl.ds(s_ * SED, SED)], pk[p].at[0],
                         psem[p])
        pltpu.async_copy(dst_hbm.at[pl.ds(s_ * SED, SED)], pk[p].at[1],
                         psem[p])
        pltpu.async_copy(et_hbm.at[pl.ds(s_ * SED, SED)], pk[p].at[2],
                         psem[p])

    def wait_packed(s_, p):
        pltpu.make_async_copy(src_hbm.at[pl.ds(s_ * SED, SED)], pk[p].at[0],
                              psem[p]).wait()
        pltpu.make_async_copy(dst_hbm.at[pl.ds(s_ * SED, SED)], pk[p].at[1],
                              psem[p]).wait()
        pltpu.make_async_copy(et_hbm.at[pl.ds(s_ * SED, SED)], pk[p].at[2],
                              psem[p]).wait()

    for p in range(2):
        fire_packed(base + p, p)

    def body(t, carry):
        for u in range(2):
            s = base + 2 * t + u
            wait_packed(s, u)
            _idx_from_packed(pk[u], gdum[u], ci[u], ddum[u])

            @pl.when(s + 2 < base + SUPW)
            def _():
                fire_packed(s + 2, u)
            for j in range(SUB):
                pltpu.sync_copy(ones_v, acc.at[ci[u].at[j]], add=True)
        return carry

    lax.fori_loop(0, SUPW // 2, body, 0)
    plsc.subcore_barrier()

    @pl.when(cid == 0)
    def _():
        pltpu.sync_copy(acc.at[pl.ds(sid * CNT_T, CNT_T)],
                        c0_hbm.at[pl.ds(sid * CNT_T, CNT_T)])

    @pl.when(cid == 1)
    def _():
        pltpu.sync_copy(acc.at[pl.ds(sid * CNT_T, CNT_T)],
                        c1_hbm.at[pl.ds(sid * CNT_T, CNT_T)])


def _sc_counts(srcp, dstp, etp, zeros1):
    return pl.kernel(
        _sc_counts_body,
        out_type=[jax.ShapeDtypeStruct((NRT,), jnp.float32),
                  jax.ShapeDtypeStruct((NRT,), jnp.float32)],
        mesh=_MESH,
        compiler_params=_SC_PARAMS,
        scratch_types=(
            [pltpu.VMEM((3, SED), jnp.int32)] * 2
            + [pltpu.VMEM((SUB, CH), jnp.int32)] * 6
            + [pltpu.VMEM((CH,), jnp.float32),
               pltpu.VMEM_SHARED((NRT,), jnp.float32)]
            + [pltpu.SemaphoreType.DMA] * 2
        ),
    )(srcp, dstp, etp, zeros1)


def _sc_inv_body(c0_hbm, c1_hbm, invc_hbm, c0_v, c1_v, iv_v, sem):
    wid = _wid()
    off = wid * NRT_W
    pltpu.sync_copy(c0_hbm.at[pl.ds(off, NRT_W)], c0_v)
    pltpu.sync_copy(c1_hbm.at[pl.ds(off, NRT_W)], c1_v)

    def body(k, carry):
        s = pl.ds(k * L, L)
        iv_v[s] = 1.0 / (c0_v[s] + c1_v[s])
        return carry

    lax.fori_loop(0, NRT_W // L, body, 0)
    pltpu.sync_copy(iv_v, invc_hbm.at[pl.ds(off, NRT_W)])


def _sc_inv(c0, c1):
    return pl.kernel(
        _sc_inv_body,
        out_type=[jax.ShapeDtypeStruct((NRT,), jnp.float32)],
        mesh=_MESH,
        compiler_params=_SC_PARAMS,
        scratch_types=[
            pltpu.VMEM((NRT_W,), jnp.float32),
            pltpu.VMEM((NRT_W,), jnp.float32),
            pltpu.VMEM((NRT_W,), jnp.float32),
            pltpu.SemaphoreType.DMA,
        ],
    )(c0, c1)[0]


def _sc_edge_body(src_hbm, dst_hbm, et_hbm, invc_hbm, y_hbm, zeros2_hbm,
                  p0_hbm, p1_hbm, *sc):
    pk = sc[0:4]
    g2 = sc[4:8]
    gd2 = sc[8:12]
    d2 = sc[12:16]
    w2 = sc[16:20]
    rows = sc[20:24]
    acc = sc[24]
    psem = sc[25:29]
    gsem = sc[29:33]
    cid = lax.axis_index("c")
    sid = lax.axis_index("s")
    base = jnp.where(cid == 0, sid * SUP0, NS * SUP0 + sid * SUP1)
    nsup = jnp.where(cid == 0, SUP0, SUP1)
    last = base + nsup

    pltpu.sync_copy(zeros2_hbm.at[pl.ds(sid * ACC_T, ACC_T)],
                    acc.at[pl.ds(sid * ACC_T, ACC_T)])
    plsc.subcore_barrier()

    def fire_packed(s, p):
        pltpu.async_copy(src_hbm.at[pl.ds(s * SED, SED)], pk[p].at[0],
                         psem[p])
        pltpu.async_copy(dst_hbm.at[pl.ds(s * SED, SED)], pk[p].at[1],
                         psem[p])
        pltpu.async_copy(et_hbm.at[pl.ds(s * SED, SED)], pk[p].at[2],
                         psem[p])

    def wait_packed(s, p):
        pltpu.make_async_copy(src_hbm.at[pl.ds(s * SED, SED)], pk[p].at[0],
                              psem[p]).wait()
        pltpu.make_async_copy(dst_hbm.at[pl.ds(s * SED, SED)], pk[p].at[1],
                              psem[p]).wait()
        pltpu.make_async_copy(et_hbm.at[pl.ds(s * SED, SED)], pk[p].at[2],
                              psem[p]).wait()

    def fire_gathers(p):
        for j in range(SUB):
            pltpu.async_copy(y_hbm.at[g2[p].at[j]],
                             rows[p].at[pl.ds(j * CH, CH)], gsem[p])
            pltpu.async_copy(invc_hbm.at[gd2[p].at[j]], w2[p].at[j], gsem[p])

    def drain_gathers(p):
        for j in range(SUB):
            pltpu.make_async_copy(y_hbm.at[g2[p].at[j]],
                                  rows[p].at[pl.ds(j * CH, CH)],
                                  gsem[p]).wait()
            pltpu.make_async_copy(invc_hbm.at[gd2[p].at[j]], w2[p].at[j],
                                  gsem[p]).wait()

    def scale(p):
        rp, wp = rows[p], w2[p]
        for j in range(SUB):
            def kb(k, carry, _j=j):
                w16 = wp[_j, pl.ds(k * L, L)]
                for l in range(L):
                    ws = w16[l]
                    ri = _j * CH + k * L + l
                    for cc in range(H // L):
                        s2 = pl.ds(cc * L, L)
                        rp[ri, s2] = rp[ri, s2] * ws
                return carry
            lax.fori_loop(0, CH // L, kb, 0)

    def fire_scatters(p):
        for j in range(SUB):
            pltpu.sync_copy(rows[p].at[pl.ds(j * CH, CH)],
                            acc.at[d2[p].at[j]], add=True)

    for p in range(4):
        fire_packed(base + p, p)
    wait_packed(base, 0)
    _idx_from_packed(pk[0], g2[0], gd2[0], d2[0])
    fire_gathers(0)
    fire_packed(base + 4, 0)

    def body(t, carry):
        for u in range(4):
            s = base + 4 * t + u
            pn = (u + 1) % 4
            sn = s + 1

            @pl.when(sn < last)
            def _():
                wait_packed(sn, pn)
                _idx_from_packed(pk[pn], g2[pn], gd2[pn], d2[pn])
                fire_gathers(pn)

                @pl.when(sn + 4 < last)
                def _():
                    fire_packed(sn + 4, pn)

            drain_gathers(u)
            scale(u)
            fire_scatters(u)
        return carry

    lax.fori_loop(0, nsup // 4, body, 0)
    plsc.subcore_barrier()

    @pl.when(cid == 0)
    def _():
        pltpu.sync_copy(acc.at[pl.ds(sid * ACC_T, ACC_T)],
                        p0_hbm.at[pl.ds(sid * ACC_T, ACC_T)])

    @pl.when(cid == 1)
    def _():
        pltpu.sync_copy(acc.at[pl.ds(sid * ACC_T, ACC_T)],
                        p1_hbm.at[pl.ds(sid * ACC_T, ACC_T)])


def _sc_edge(srcp, dstp, etp, invc, y, zeros2):
    return pl.kernel(
        _sc_edge_body,
        out_type=[jax.ShapeDtypeStruct((NACC, H), jnp.float32),
                  jax.ShapeDtypeStruct((NACC, H), jnp.float32)],
        mesh=_MESH,
        compiler_params=_SC_PARAMS,
        scratch_types=(
            [pltpu.VMEM((3, SED), jnp.int32)] * 4
            + [pltpu.VMEM((SUB, CH), jnp.int32)] * 12
            + [pltpu.VMEM((SUB, CH), jnp.float32)] * 4
            + [pltpu.VMEM((SED, H), jnp.float32)] * 4
            + [pltpu.VMEM_SHARED((NACC, H), jnp.float32)]
            + [pltpu.SemaphoreType.DMA] * 8
        ),
    )(srcp, dstp, etp, invc, y, zeros2)


def kernel(feature, edge_index, edge_type, Wd, bd, Wt, bt, Wn, bn, Wc, bc,
           Wi, bi, W_rel, W_root, b_r, Wo1, bo1, Wo2, bo2):
    f32 = jnp.float32
    i32 = jnp.int32
    wenc = jnp.zeros((FEAT, H), f32)
    wenc = wenc.at[46:814, 0:16].set(Wd)
    wenc = wenc.at[814:1582, 16:32].set(Wt)
    wenc = wenc.at[12:46, 32:48].set(Wn)
    wenc = wenc.at[0:12, 48:64].set(Wc)
    benc = jnp.concatenate([bd, bt, bn, bc]).reshape(1, H)
    bi2 = bi.reshape(1, H)
    br2 = b_r.reshape(1, H)
    bo12 = bo1.reshape(1, H)
    wrelf = jnp.transpose(W_rel, (1, 0, 2)).reshape(H, R * H)
    wo2p = jnp.zeros((H, 128), f32).at[:, :2].set(Wo2)
    bo2p = jnp.zeros((1, 128), f32).at[0, :2].set(bo2)

    srcp = jnp.concatenate([edge_index[0], jnp.zeros((PAD,), i32)])
    dstp = jnp.concatenate([edge_index[1], jnp.full((PAD,), N, i32)])
    etp = jnp.concatenate([edge_type, jnp.zeros((PAD,), i32)])
    zeros1 = jnp.zeros((NRT,), f32)
    zeros2 = jnp.zeros((NACC, H), f32)

    y1, root1 = _tc_encoder(feature.T, wenc, benc, Wi, bi2, wrelf, W_root,
                            br2)

    c0, c1 = _sc_counts(srcp, dstp, etp, zeros1)
    invc = _sc_inv(c0, c1)

    p0, p1 = _sc_edge(srcp, dstp, etp, invc, y1.reshape(NR, H), zeros2)

    y2, root2 = _tc_mid(p0[:N], p1[:N], root1, wrelf, W_root, br2)

    q0, q1 = _sc_edge(srcp, dstp, etp, invc, y2.reshape(NR, H), zeros2)

    out = _tc_out(q0[:N], q1[:N], root2, Wo1, bo12, wo2p, bo2p)
    return out[:, :2]
